# baseline (speedup 1.0000x reference)
import jax
import jax.numpy as jnp
from jax import lax
from jax.experimental import pallas as pl
from jax.experimental.pallas import tpu as pltpu
from jax.experimental.pallas import tpu_sc as plsc

NC = 2
NS = 16
NW = NC * NS
L = 16
CHUNK = 128


def _make_sc_pass(npad, d, k0, k1, chunk):
    mesh = plsc.VectorSubcoreMesh(core_axis_name="c", subcore_axis_name="s",
                                  num_cores=NC, num_subcores=NS)
    rpt = npad // NS
    kmax = max(k0, k1)

    def body(h_hbm, src_a, dst_a, *rest):
        if k1:
            src_b, dst_b, psum_hbm, src_v, dst_v, rows_v, acc_sh, gsem0, \
                gsem1 = rest
        else:
            psum_hbm, src_v, dst_v, rows_v, acc_sh, gsem0, gsem1 = rest
        c = lax.axis_index("c")
        s = lax.axis_index("s")

        b0 = rows_v.at[0]
        b1 = rows_v.at[1]
        zeros16 = jnp.zeros((L,), jnp.float32)

        def zrow(i, carry):
            for t in range(d // L):
                rows_v[0, i, pl.ds(t * L, L)] = zeros16
            return carry

        lax.fori_loop(0, chunk, zrow, 0)
        for r in range(rpt // chunk):
            pltpu.sync_copy(b0, acc_sh.at[pl.ds(s * rpt + r * chunk, chunk)])

        plsc.subcore_barrier()

        def pipeline(src_h, dst_h, kx):
            hk = kx // 4
            for hh in range(4):
                pltpu.sync_copy(src_h.at[s, pl.ds(hh * hk, hk)],
                                src_v.at[pl.ds(0, hk)])
                pltpu.sync_copy(dst_h.at[s, pl.ds(hh * hk, hk)],
                                dst_v.at[pl.ds(0, hk)])

                def step(j, carry):
                    pltpu.async_copy(h_hbm.at[src_v.at[j]], b0, gsem0).wait()
                    pltpu.sync_copy(b0, acc_sh.at[dst_v.at[j]], add=True)
                    return carry

                lax.fori_loop(0, hk, step, 0)

        @pl.when(c == 0)
        def _():
            pipeline(src_a, dst_a, k0)

        if k1:
            @pl.when(c == 1)
            def _():
                pipeline(src_b, dst_b, k1)

        plsc.subcore_barrier()

        @pl.when(jnp.logical_or(c == 0, bool(k1)))
        def _():
            pltpu.sync_copy(acc_sh.at[pl.ds(s * rpt, rpt)],
                            psum_hbm.at[c, pl.ds(s * rpt, rpt)])

    ncp = 2 if k1 else 1
    return pl.kernel(
        body,
        out_type=jax.ShapeDtypeStruct((ncp, npad, d), jnp.float32),
        mesh=mesh,
        scratch_types=(
            pltpu.VMEM((kmax // 4, chunk), jnp.int32),
            pltpu.VMEM((kmax // 4, chunk), jnp.int32),
            pltpu.VMEM((2, chunk, d), jnp.float32),
            pltpu.VMEM_SHARED((npad, d), jnp.float32),
            pltpu.SemaphoreType.DMA,
            pltpu.SemaphoreType.DMA,
        ),
    )


def _make_sc_count_pass(npad, k_chunks):
    mesh = plsc.VectorSubcoreMesh(core_axis_name="c", subcore_axis_name="s",
                                  num_cores=NC, num_subcores=NS)
    rpt = npad // NS

    def body(dst_hbm, cnt_hbm, dst_v, ones_v, zero_v, cnt_sh):
        c = lax.axis_index("c")
        s = lax.axis_index("s")
        wid = s * NC + c

        def fill(i, carry):
            ones_v[i, pl.ds(0, L)] = jnp.full((L,), 1.0, jnp.float32)
            zero_v[i, pl.ds(0, L)] = jnp.zeros((L,), jnp.float32)
            return carry

        lax.fori_loop(0, CHUNK, fill, 0)
        for r in range(rpt // CHUNK):
            pltpu.sync_copy(zero_v, cnt_sh.at[pl.ds(s * rpt + r * CHUNK, CHUNK)])
        pltpu.sync_copy(dst_hbm.at[wid], dst_v)

        plsc.subcore_barrier()

        def step(j, carry):
            pltpu.sync_copy(ones_v, cnt_sh.at[dst_v.at[j]], add=True)
            return carry

        lax.fori_loop(0, k_chunks, step, 0)

        plsc.subcore_barrier()

        pltpu.sync_copy(cnt_sh.at[pl.ds(s * rpt, rpt)],
                        cnt_hbm.at[c, pl.ds(s * rpt, rpt)])

    return pl.kernel(
        body,
        out_type=jax.ShapeDtypeStruct((NC, npad, L), jnp.float32),
        mesh=mesh,
        compiler_params=pltpu.CompilerParams(use_tc_tiling_on_sc=False),
        scratch_types=(
            pltpu.VMEM((k_chunks, CHUNK), jnp.int32),
            pltpu.VMEM((CHUNK, L), jnp.float32),
            pltpu.VMEM((CHUNK, L), jnp.float32),
            pltpu.VMEM_SHARED((npad, L), jnp.float32),
        ),
    )


def _make_tc_layer1(npad, d, blk, ncp):
    def body(p_ref, c_ref, h_ref, wl_ref, bl_ref, wr_ref, out_ref, cnt_ref):
        ssum = p_ref[0] + p_ref[1] if ncp == 2 else p_ref[0]
        cnt = jnp.maximum((c_ref[0] + c_ref[1])[:, 0:1], 1.0)
        agg = ssum / cnt
        out = (jnp.dot(agg, wl_ref[...], preferred_element_type=jnp.float32)
               + bl_ref[...]
               + jnp.dot(h_ref[...], wr_ref[...],
                         preferred_element_type=jnp.float32))
        out_ref[...] = jnp.maximum(out, 0.0)
        cnt_ref[...] = cnt

    return pl.pallas_call(
        body,
        grid=(npad // blk,),
        in_specs=[
            pl.BlockSpec((ncp, blk, d), lambda i: (0, i, 0)),
            pl.BlockSpec((NC, blk, L), lambda i: (0, i, 0)),
            pl.BlockSpec((blk, d), lambda i: (i, 0)),
            pl.BlockSpec((d, d), lambda i: (0, 0)),
            pl.BlockSpec((1, d), lambda i: (0, 0)),
            pl.BlockSpec((d, d), lambda i: (0, 0)),
        ],
        out_specs=[
            pl.BlockSpec((blk, d), lambda i: (i, 0)),
            pl.BlockSpec((blk, 1), lambda i: (i, 0)),
        ],
        out_shape=[
            jax.ShapeDtypeStruct((npad, d), jnp.float32),
            jax.ShapeDtypeStruct((npad, 1), jnp.float32),
        ],
    )


def _make_tc_layer2(npad, d, blk, ncp):
    def body(p_ref, cnt_ref, h_ref, wl_ref, bl_ref, wr_ref, out_ref):
        ssum = p_ref[0] + p_ref[1] if ncp == 2 else p_ref[0]
        agg = ssum / cnt_ref[...]
        out_ref[...] = (
            jnp.dot(agg, wl_ref[...], preferred_element_type=jnp.float32)
            + bl_ref[...]
            + jnp.dot(h_ref[...], wr_ref[...],
                      preferred_element_type=jnp.float32))

    return pl.pallas_call(
        body,
        grid=(npad // blk,),
        in_specs=[
            pl.BlockSpec((ncp, blk, d), lambda i: (0, i, 0)),
            pl.BlockSpec((blk, 1), lambda i: (i, 0)),
            pl.BlockSpec((blk, d), lambda i: (i, 0)),
            pl.BlockSpec((d, d), lambda i: (0, 0)),
            pl.BlockSpec((1, d), lambda i: (0, 0)),
            pl.BlockSpec((d, d), lambda i: (0, 0)),
        ],
        out_specs=pl.BlockSpec((blk, d), lambda i: (i, 0)),
        out_shape=jax.ShapeDtypeStruct((npad, d), jnp.float32),
    )


FCHUNK = 128
SPLIT0 = 0.61


@jax.jit
def kernel(x, edge_index, Wl0, bl0, Wr0, Wl1, bl1, Wr1):
    n, d = x.shape
    e = edge_index.shape[1]

    npad = (n // (NS * CHUNK) + 1) * (NS * CHUNK)

    kc = -(-e // (CHUNK * NW))
    epad_c = kc * CHUNK * NW
    dst3 = jnp.pad(edge_index[1], (0, epad_c - e),
                   constant_values=npad - 1).reshape(NW, kc, CHUNK)

    chunks = -(-e // FCHUNK)
    k0 = -(--(-int(chunks * SPLIT0) // NS) // 32) * 32
    if k0 * NS >= chunks:
        k1 = 0
    else:
        k1 = -(-max(32, -(-(chunks - k0 * NS) // NS)) // 32) * 32
    epad = (k0 + k1) * NS * FCHUNK
    srcp = jnp.pad(edge_index[0], (0, epad - e))
    dstp = jnp.pad(edge_index[1], (0, epad - e), constant_values=npad - 1)
    na = k0 * NS * FCHUNK
    src_a = srcp[:na].reshape(NS, k0, FCHUNK)
    dst_a = dstp[:na].reshape(NS, k0, FCHUNK)

    xp = jnp.pad(x, ((0, npad - n), (0, 0)))

    blk = 512
    ncp = 2 if k1 else 1
    sc_feat = _make_sc_pass(npad, d, k0, k1, FCHUNK)
    if k1:
        src_b = srcp[na:].reshape(NS, k1, FCHUNK)
        dst_b = dstp[na:].reshape(NS, k1, FCHUNK)
        feat_args = (src_a, dst_a, src_b, dst_b)
    else:
        feat_args = (src_a, dst_a)
    cntp = _make_sc_count_pass(npad, kc)(dst3)
    psum0 = sc_feat(xp, *feat_args)
    h1, cnt = _make_tc_layer1(npad, d, blk, ncp)(
        psum0, cntp, xp, Wl0, bl0[None, :], Wr0)
    psum1 = sc_feat(h1, *feat_args)
    out = _make_tc_layer2(npad, d, blk, ncp)(
        psum1, cnt, h1, Wl1, bl1[None, :], Wr1)
    return out[:n]


# speedup vs baseline: 1.0002x; 1.0002x over previous
import jax
import jax.numpy as jnp
from jax import lax
from jax.experimental import pallas as pl
from jax.experimental.pallas import tpu as pltpu
from jax.experimental.pallas import tpu_sc as plsc

NC = 2
NS = 16
NW = NC * NS
L = 16
CHUNK = 128


def _make_sc_pass(npad, d, k0, k1, chunk):
    mesh = plsc.VectorSubcoreMesh(core_axis_name="c", subcore_axis_name="s",
                                  num_cores=NC, num_subcores=NS)
    rpt = npad // NS
    kmax = max(k0, k1)

    def body(h_hbm, src_a, dst_a, *rest):
        if k1:
            src_b, dst_b, psum_hbm, src_v, dst_v, rows_v, acc_sh, gsem0, \
                gsem1 = rest
        else:
            psum_hbm, src_v, dst_v, rows_v, acc_sh, gsem0, gsem1 = rest
        c = lax.axis_index("c")
        s = lax.axis_index("s")

        b0 = rows_v.at[0]
        b1 = rows_v.at[1]
        zeros16 = jnp.zeros((L,), jnp.float32)

        def zrow(i, carry):
            for t in range(d // L):
                rows_v[0, i, pl.ds(t * L, L)] = zeros16
            return carry

        lax.fori_loop(0, chunk, zrow, 0)
        for r in range(rpt // chunk):
            pltpu.sync_copy(b0, acc_sh.at[pl.ds(s * rpt + r * chunk, chunk)])

        plsc.subcore_barrier()

        def pipeline(src_h, dst_h, kx):
            hk = kx // 4
            for hh in range(4):
                pltpu.sync_copy(src_h.at[s, pl.ds(hh * hk, hk)],
                                src_v.at[pl.ds(0, hk)])
                pltpu.sync_copy(dst_h.at[s, pl.ds(hh * hk, hk)],
                                dst_v.at[pl.ds(0, hk)])

                def step(j, carry):
                    pltpu.async_copy(h_hbm.at[src_v.at[j]], b0, gsem0).wait()
                    pltpu.sync_copy(b0, acc_sh.at[dst_v.at[j]], add=True)
                    return carry

                lax.fori_loop(0, hk, step, 0)

        @pl.when(c == 0)
        def _():
            pipeline(src_a, dst_a, k0)

        if k1:
            @pl.when(c == 1)
            def _():
                pipeline(src_b, dst_b, k1)

        plsc.subcore_barrier()

        @pl.when(jnp.logical_or(c == 0, bool(k1)))
        def _():
            pltpu.sync_copy(acc_sh.at[pl.ds(s * rpt, rpt)],
                            psum_hbm.at[c, pl.ds(s * rpt, rpt)])

    ncp = 2 if k1 else 1
    return pl.kernel(
        body,
        out_type=jax.ShapeDtypeStruct((ncp, npad, d), jnp.float32),
        mesh=mesh,
        scratch_types=(
            pltpu.VMEM((kmax // 4, chunk), jnp.int32),
            pltpu.VMEM((kmax // 4, chunk), jnp.int32),
            pltpu.VMEM((2, chunk, d), jnp.float32),
            pltpu.VMEM_SHARED((npad, d), jnp.float32),
            pltpu.SemaphoreType.DMA,
            pltpu.SemaphoreType.DMA,
        ),
    )


def _make_sc_count_pass(npad, k_chunks):
    mesh = plsc.VectorSubcoreMesh(core_axis_name="c", subcore_axis_name="s",
                                  num_cores=NC, num_subcores=NS)
    rpt = npad // NS

    def body(dst_hbm, cnt_hbm, dst_v, ones_v, zero_v, cnt_sh):
        c = lax.axis_index("c")
        s = lax.axis_index("s")
        wid = s * NC + c

        def fill(i, carry):
            ones_v[i, pl.ds(0, L)] = jnp.full((L,), 1.0, jnp.float32)
            zero_v[i, pl.ds(0, L)] = jnp.zeros((L,), jnp.float32)
            return carry

        lax.fori_loop(0, CHUNK, fill, 0)
        for r in range(rpt // CHUNK):
            pltpu.sync_copy(zero_v, cnt_sh.at[pl.ds(s * rpt + r * CHUNK, CHUNK)])
        pltpu.sync_copy(dst_hbm.at[wid], dst_v)

        plsc.subcore_barrier()

        def step(j, carry):
            pltpu.sync_copy(ones_v, cnt_sh.at[dst_v.at[j]], add=True)
            return carry

        lax.fori_loop(0, k_chunks, step, 0)

        plsc.subcore_barrier()

        pltpu.sync_copy(cnt_sh.at[pl.ds(s * rpt, rpt)],
                        cnt_hbm.at[c, pl.ds(s * rpt, rpt)])

    return pl.kernel(
        body,
        out_type=jax.ShapeDtypeStruct((NC, npad, L), jnp.float32),
        mesh=mesh,
        compiler_params=pltpu.CompilerParams(use_tc_tiling_on_sc=False),
        scratch_types=(
            pltpu.VMEM((k_chunks, CHUNK), jnp.int32),
            pltpu.VMEM((CHUNK, L), jnp.float32),
            pltpu.VMEM((CHUNK, L), jnp.float32),
            pltpu.VMEM_SHARED((npad, L), jnp.float32),
        ),
    )


def _make_tc_layer1(npad, d, blk, ncp):
    def body(p_ref, c_ref, h_ref, wl_ref, bl_ref, wr_ref, out_ref, cnt_ref):
        ssum = p_ref[0] + p_ref[1] if ncp == 2 else p_ref[0]
        cnt = jnp.maximum((c_ref[0] + c_ref[1])[:, 0:1], 1.0)
        agg = ssum / cnt
        out = (jnp.dot(agg, wl_ref[...], preferred_element_type=jnp.float32)
               + bl_ref[...]
               + jnp.dot(h_ref[...], wr_ref[...],
                         preferred_element_type=jnp.float32))
        out_ref[...] = jnp.maximum(out, 0.0)
        cnt_ref[...] = cnt

    return pl.pallas_call(
        body,
        grid=(npad // blk,),
        in_specs=[
            pl.BlockSpec((ncp, blk, d), lambda i: (0, i, 0)),
            pl.BlockSpec((NC, blk, L), lambda i: (0, i, 0)),
            pl.BlockSpec((blk, d), lambda i: (i, 0)),
            pl.BlockSpec((d, d), lambda i: (0, 0)),
            pl.BlockSpec((1, d), lambda i: (0, 0)),
            pl.BlockSpec((d, d), lambda i: (0, 0)),
        ],
        out_specs=[
            pl.BlockSpec((blk, d), lambda i: (i, 0)),
            pl.BlockSpec((blk, 1), lambda i: (i, 0)),
        ],
        out_shape=[
            jax.ShapeDtypeStruct((npad, d), jnp.float32),
            jax.ShapeDtypeStruct((npad, 1), jnp.float32),
        ],
    )


def _make_tc_layer2(npad, d, blk, ncp):
    def body(p_ref, cnt_ref, h_ref, wl_ref, bl_ref, wr_ref, out_ref):
        ssum = p_ref[0] + p_ref[1] if ncp == 2 else p_ref[0]
        agg = ssum / cnt_ref[...]
        out_ref[...] = (
            jnp.dot(agg, wl_ref[...], preferred_element_type=jnp.float32)
            + bl_ref[...]
            + jnp.dot(h_ref[...], wr_ref[...],
                      preferred_element_type=jnp.float32))

    return pl.pallas_call(
        body,
        grid=(npad // blk,),
        in_specs=[
            pl.BlockSpec((ncp, blk, d), lambda i: (0, i, 0)),
            pl.BlockSpec((blk, 1), lambda i: (i, 0)),
            pl.BlockSpec((blk, d), lambda i: (i, 0)),
            pl.BlockSpec((d, d), lambda i: (0, 0)),
            pl.BlockSpec((1, d), lambda i: (0, 0)),
            pl.BlockSpec((d, d), lambda i: (0, 0)),
        ],
        out_specs=pl.BlockSpec((blk, d), lambda i: (i, 0)),
        out_shape=jax.ShapeDtypeStruct((npad, d), jnp.float32),
    )


FCHUNK = 128
SPLIT0 = 0.5


@jax.jit
def kernel(x, edge_index, Wl0, bl0, Wr0, Wl1, bl1, Wr1):
    n, d = x.shape
    e = edge_index.shape[1]

    npad = (n // (NS * CHUNK) + 1) * (NS * CHUNK)

    kc = -(-e // (CHUNK * NW))
    epad_c = kc * CHUNK * NW
    dst3 = jnp.pad(edge_index[1], (0, epad_c - e),
                   constant_values=npad - 1).reshape(NW, kc, CHUNK)

    chunks = -(-e // FCHUNK)
    k0 = -(--(-int(chunks * SPLIT0) // NS) // 32) * 32
    if k0 * NS >= chunks:
        k1 = 0
    else:
        k1 = -(-max(32, -(-(chunks - k0 * NS) // NS)) // 32) * 32
    epad = (k0 + k1) * NS * FCHUNK
    srcp = jnp.pad(edge_index[0], (0, epad - e))
    dstp = jnp.pad(edge_index[1], (0, epad - e), constant_values=npad - 1)
    na = k0 * NS * FCHUNK
    src_a = srcp[:na].reshape(NS, k0, FCHUNK)
    dst_a = dstp[:na].reshape(NS, k0, FCHUNK)

    xp = jnp.pad(x, ((0, npad - n), (0, 0)))

    blk = 512
    ncp = 2 if k1 else 1
    sc_feat = _make_sc_pass(npad, d, k0, k1, FCHUNK)
    if k1:
        src_b = srcp[na:].reshape(NS, k1, FCHUNK)
        dst_b = dstp[na:].reshape(NS, k1, FCHUNK)
        feat_args = (src_a, dst_a, src_b, dst_b)
    else:
        feat_args = (src_a, dst_a)
    cntp = _make_sc_count_pass(npad, kc)(dst3)
    psum0 = sc_feat(xp, *feat_args)
    h1, cnt = _make_tc_layer1(npad, d, blk, ncp)(
        psum0, cntp, xp, Wl0, bl0[None, :], Wr0)
    psum1 = sc_feat(h1, *feat_args)
    out = _make_tc_layer2(npad, d, blk, ncp)(
        psum1, cnt, h1, Wl1, bl1[None, :], Wr1)
    return out[:n]


# speedup vs baseline: 1.8779x; 1.8775x over previous
import jax
import jax.numpy as jnp
from jax import lax
from jax.experimental import pallas as pl
from jax.experimental.pallas import tpu as pltpu
from jax.experimental.pallas import tpu_sc as plsc

NC = 2
NS = 16
NW = NC * NS
L = 16
CHUNK = 128


def _make_sc_pass(npad, d, k0, k1, kmax, chunk):
    mesh = plsc.VectorSubcoreMesh(core_axis_name="c", subcore_axis_name="s",
                                  num_cores=NC, num_subcores=NS)
    rpt = npad // NS
    hk = kmax // 4

    def body(h_hbm, src4, dst4, psum_hbm, src_v, dst_v, rows_v, acc_sh,
             gsem0, gsem1):
        c = lax.axis_index("c")
        s = lax.axis_index("s")
        kc = jnp.where(c == 0, k0, k1)

        b0 = rows_v.at[0]
        zeros16 = jnp.zeros((L,), jnp.float32)

        def zrow(i, carry):
            for t in range(d // L):
                rows_v[0, i, pl.ds(t * L, L)] = zeros16
            return carry

        lax.fori_loop(0, chunk, zrow, 0)
        for r in range(rpt // chunk):
            pltpu.sync_copy(b0, acc_sh.at[pl.ds(s * rpt + r * chunk, chunk)])

        plsc.subcore_barrier()

        for hh in range(4):
            pltpu.sync_copy(src4.at[c, s, pl.ds(hh * hk, hk)],
                            src_v.at[pl.ds(0, hk)])
            pltpu.sync_copy(dst4.at[c, s, pl.ds(hh * hk, hk)],
                            dst_v.at[pl.ds(0, hk)])

            def step(j, carry):
                pltpu.async_copy(h_hbm.at[src_v.at[j]], b0, gsem0).wait()
                pltpu.sync_copy(b0, acc_sh.at[dst_v.at[j]], add=True)
                return carry

            nj = jnp.clip(kc - hh * hk, 0, hk)
            lax.fori_loop(0, nj, step, 0)

        plsc.subcore_barrier()

        pltpu.sync_copy(acc_sh.at[pl.ds(s * rpt, rpt)],
                        psum_hbm.at[c, pl.ds(s * rpt, rpt)])

    return pl.kernel(
        body,
        out_type=jax.ShapeDtypeStruct((NC, npad, d), jnp.float32),
        mesh=mesh,
        scratch_types=(
            pltpu.VMEM((hk, chunk), jnp.int32),
            pltpu.VMEM((hk, chunk), jnp.int32),
            pltpu.VMEM((2, chunk, d), jnp.float32),
            pltpu.VMEM_SHARED((npad, d), jnp.float32),
            pltpu.SemaphoreType.DMA,
            pltpu.SemaphoreType.DMA,
        ),
    )


def _make_sc_count_pass(npad, k_chunks):
    mesh = plsc.VectorSubcoreMesh(core_axis_name="c", subcore_axis_name="s",
                                  num_cores=NC, num_subcores=NS)
    rpt = npad // NS

    def body(dst_hbm, cnt_hbm, dst_v, ones_v, zero_v, cnt_sh):
        c = lax.axis_index("c")
        s = lax.axis_index("s")
        wid = s * NC + c

        def fill(i, carry):
            ones_v[i, pl.ds(0, L)] = jnp.full((L,), 1.0, jnp.float32)
            zero_v[i, pl.ds(0, L)] = jnp.zeros((L,), jnp.float32)
            return carry

        lax.fori_loop(0, CHUNK, fill, 0)
        for r in range(rpt // CHUNK):
            pltpu.sync_copy(zero_v, cnt_sh.at[pl.ds(s * rpt + r * CHUNK, CHUNK)])
        pltpu.sync_copy(dst_hbm.at[wid], dst_v)

        plsc.subcore_barrier()

        def step(j, carry):
            pltpu.sync_copy(ones_v, cnt_sh.at[dst_v.at[j]], add=True)
            return carry

        lax.fori_loop(0, k_chunks, step, 0)

        plsc.subcore_barrier()

        pltpu.sync_copy(cnt_sh.at[pl.ds(s * rpt, rpt)],
                        cnt_hbm.at[c, pl.ds(s * rpt, rpt)])

    return pl.kernel(
        body,
        out_type=jax.ShapeDtypeStruct((NC, npad, L), jnp.float32),
        mesh=mesh,
        compiler_params=pltpu.CompilerParams(use_tc_tiling_on_sc=False),
        scratch_types=(
            pltpu.VMEM((k_chunks, CHUNK), jnp.int32),
            pltpu.VMEM((CHUNK, L), jnp.float32),
            pltpu.VMEM((CHUNK, L), jnp.float32),
            pltpu.VMEM_SHARED((npad, L), jnp.float32),
        ),
    )


def _make_tc_layer1(npad, d, blk, ncp):
    def body(p_ref, c_ref, h_ref, wl_ref, bl_ref, wr_ref, out_ref, cnt_ref):
        ssum = p_ref[0] + p_ref[1] if ncp == 2 else p_ref[0]
        cnt = jnp.maximum((c_ref[0] + c_ref[1])[:, 0:1], 1.0)
        agg = ssum / cnt
        out = (jnp.dot(agg, wl_ref[...], preferred_element_type=jnp.float32)
               + bl_ref[...]
               + jnp.dot(h_ref[...], wr_ref[...],
                         preferred_element_type=jnp.float32))
        out_ref[...] = jnp.maximum(out, 0.0)
        cnt_ref[...] = cnt

    return pl.pallas_call(
        body,
        grid=(npad // blk,),
        in_specs=[
            pl.BlockSpec((ncp, blk, d), lambda i: (0, i, 0)),
            pl.BlockSpec((NC, blk, L), lambda i: (0, i, 0)),
            pl.BlockSpec((blk, d), lambda i: (i, 0)),
            pl.BlockSpec((d, d), lambda i: (0, 0)),
            pl.BlockSpec((1, d), lambda i: (0, 0)),
            pl.BlockSpec((d, d), lambda i: (0, 0)),
        ],
        out_specs=[
            pl.BlockSpec((blk, d), lambda i: (i, 0)),
            pl.BlockSpec((blk, 1), lambda i: (i, 0)),
        ],
        out_shape=[
            jax.ShapeDtypeStruct((npad, d), jnp.float32),
            jax.ShapeDtypeStruct((npad, 1), jnp.float32),
        ],
    )


def _make_tc_layer2(npad, d, blk, ncp):
    def body(p_ref, cnt_ref, h_ref, wl_ref, bl_ref, wr_ref, out_ref):
        ssum = p_ref[0] + p_ref[1] if ncp == 2 else p_ref[0]
        agg = ssum / cnt_ref[...]
        out_ref[...] = (
            jnp.dot(agg, wl_ref[...], preferred_element_type=jnp.float32)
            + bl_ref[...]
            + jnp.dot(h_ref[...], wr_ref[...],
                      preferred_element_type=jnp.float32))

    return pl.pallas_call(
        body,
        grid=(npad // blk,),
        in_specs=[
            pl.BlockSpec((ncp, blk, d), lambda i: (0, i, 0)),
            pl.BlockSpec((blk, 1), lambda i: (i, 0)),
            pl.BlockSpec((blk, d), lambda i: (i, 0)),
            pl.BlockSpec((d, d), lambda i: (0, 0)),
            pl.BlockSpec((1, d), lambda i: (0, 0)),
            pl.BlockSpec((d, d), lambda i: (0, 0)),
        ],
        out_specs=pl.BlockSpec((blk, d), lambda i: (i, 0)),
        out_shape=jax.ShapeDtypeStruct((npad, d), jnp.float32),
    )


FCHUNK = 128
SPLIT0 = 0.5


@jax.jit
def kernel(x, edge_index, Wl0, bl0, Wr0, Wl1, bl1, Wr1):
    n, d = x.shape
    e = edge_index.shape[1]

    npad = (n // (NS * CHUNK) + 1) * (NS * CHUNK)

    kc = -(-e // (CHUNK * NW))
    epad_c = kc * CHUNK * NW
    dst3 = jnp.pad(edge_index[1], (0, epad_c - e),
                   constant_values=npad - 1).reshape(NW, kc, CHUNK)

    chunks = -(-e // FCHUNK)
    kmax = 128
    k0 = min(kmax, -(-int(chunks * SPLIT0) // NS))
    k1 = min(kmax, max(0, -(-(chunks - k0 * NS) // NS)))
    assert (k0 + k1) * NS >= chunks

    epad = (k0 + k1) * NS * FCHUNK
    srcp = jnp.pad(edge_index[0], (0, epad - e))
    dstp = jnp.pad(edge_index[1], (0, epad - e), constant_values=npad - 1)
    na = k0 * NS * FCHUNK

    def to4(arr, fill):
        a = arr[:na].reshape(NS, k0, FCHUNK)
        b = arr[na:].reshape(NS, k1, FCHUNK)
        a = jnp.pad(a, ((0, 0), (0, kmax - k0), (0, 0)), constant_values=fill)
        b = jnp.pad(b, ((0, 0), (0, kmax - k1), (0, 0)), constant_values=fill)
        return jnp.stack([a, b])

    src4 = to4(srcp, 0)
    dst4 = to4(dstp, npad - 1)

    xp = jnp.pad(x, ((0, npad - n), (0, 0)))

    blk = 512
    sc_feat = _make_sc_pass(npad, d, k0, k1, kmax, FCHUNK)
    cntp = _make_sc_count_pass(npad, kc)(dst3)
    psum0 = sc_feat(xp, src4, dst4)
    h1, cnt = _make_tc_layer1(npad, d, blk, NC)(
        psum0, cntp, xp, Wl0, bl0[None, :], Wr0)
    psum1 = sc_feat(h1, src4, dst4)
    out = _make_tc_layer2(npad, d, blk, NC)(
        psum1, cnt, h1, Wl1, bl1[None, :], Wr1)
    return out[:n]
